# baseline (speedup 1.0000x reference)
import functools

import jax
import jax.numpy as jnp
from jax import lax
from jax.experimental import pallas as pl
from jax.experimental.pallas import tpu as pltpu
from jax.experimental.pallas import tpu_sc as plsc

N = 10000
NPAD = 10240
D = 128
DH = D // 2
DO = 64
E = 320000
NC, NS = 2, 16
EPT = E // NS
CH = 80
NCH = EPT // CH
RPT = NPAD // NS
CW = 16
BR = 1000
NBUF = 5
LEAD = 2
DRAIN = NBUF - LEAD


def _fill2d(ref, rows, cols, value):
  vecs = cols // 16

  def body(t, carry):
    r = t // vecs
    c = (t % vecs) * 16
    ref[r, pl.ds(c, 16)] = jnp.full((16,), value, jnp.float32)
    return carry

  lax.fori_loop(0, rows * vecs, body, 0)


def _sc_segment_sum(table2, src2, dst3):
  mesh = plsc.VectorSubcoreMesh(core_axis_name="c", subcore_axis_name="s")

  @functools.partial(
      pl.kernel,
      out_type=[
          jax.ShapeDtypeStruct((NC, NPAD, DH), jnp.float32),
          jax.ShapeDtypeStruct((NC, NPAD, CW), jnp.float32),
      ],
      mesh=mesh,
      compiler_params=pltpu.CompilerParams(use_tc_tiling_on_sc=False),
      scratch_types=[
          pltpu.VMEM((EPT,), jnp.int32),
          pltpu.VMEM((NCH, CH), jnp.int32),
          pltpu.VMEM((NBUF, CH, DH), jnp.float32),
          pltpu.VMEM((CH, CW), jnp.float32),
          pltpu.VMEM((CH, CW), jnp.float32),
          pltpu.VMEM_SHARED((NPAD, DH), jnp.float32),
          pltpu.VMEM_SHARED((NPAD, CW), jnp.float32),
          pltpu.SemaphoreType.DMA((NBUF,)),
          pltpu.SemaphoreType.DMA((NBUF,)),
          pltpu.SemaphoreType.DMA,
      ],
  )
  def k(table_h, src_h, dst_h,
        sums_h, cnts_h,
        src_v, dst_v, rows_v, ones_v, zcnt_v, acc_s, cnt_s,
        gsem, ssem, csem):
    cid = lax.axis_index("c")
    sid = lax.axis_index("s")

    pltpu.sync_copy(src_h.at[sid], src_v)
    pltpu.sync_copy(dst_h.at[sid], dst_v)
    _fill2d(rows_v.at[0], CH, DH, 0.0)
    _fill2d(ones_v, CH, CW, 1.0)
    _fill2d(zcnt_v, CH, CW, 0.0)
    base = pl.multiple_of(sid * RPT, 8)
    for kk in range(RPT // CH):
      off = pl.multiple_of(base + kk * CH, 8)
      pltpu.sync_copy(rows_v.at[0], acc_s.at[pl.ds(off, CH)])
      pltpu.sync_copy(zcnt_v, cnt_s.at[pl.ds(off, CH)])

    my_table = table_h.at[cid]

    def g_issue(j, b):
      off = pl.multiple_of(j * CH, CH)
      pltpu.async_copy(my_table.at[src_v.at[pl.ds(off, CH)]], rows_v.at[b],
                       gsem.at[b])

    def g_wait(j, b):
      off = pl.multiple_of(j * CH, CH)
      pltpu.make_async_copy(my_table.at[src_v.at[pl.ds(off, CH)]],
                            rows_v.at[b], gsem.at[b]).wait()

    def s_issue(j, b):
      pltpu.async_copy(rows_v.at[b], acc_s.at[dst_v.at[j]], ssem.at[b],
                       add=True)

    def s_wait(j, b):
      pltpu.make_async_copy(rows_v.at[b], acc_s.at[dst_v.at[j]],
                            ssem.at[b]).wait()

    def c_issue(j):
      @pl.when(jnp.logical_xor(cid == 1, j < NCH // 2))
      def _():
        pltpu.async_copy(ones_v, cnt_s.at[dst_v.at[j]], csem, add=True)

    def step(j, b):
      g_wait(j, b)
      s_issue(j, b)
      c_issue(j)

    for b in range(LEAD):
      g_issue(b, b)
    plsc.subcore_barrier()

    for b in range(NBUF):
      j = b
      step(j, b)
      if j >= DRAIN:
        s_wait(j - DRAIN, (j + LEAD) % NBUF)
      g_issue(j + LEAD, (j + LEAD) % NBUF)

    def outer(go, carry):
      j0 = go * NBUF
      for b in range(NBUF):
        j = j0 + b
        step(j, b)
        s_wait(j - DRAIN, (b + LEAD) % NBUF)
        g_issue(j + LEAD, (b + LEAD) % NBUF)
      return carry

    lax.fori_loop(1, NCH // NBUF - 1, outer, 0)

    for b in range(NBUF):
      j = NCH - NBUF + b
      step(j, b)
      if j + LEAD < NCH:
        s_wait(j - DRAIN, (b + LEAD) % NBUF)
        g_issue(j + LEAD, (b + LEAD) % NBUF)
    for b in range(NBUF):
      j = NCH - NBUF + b
      s_wait(j, b)

    def c_drain(i, carry):
      pltpu.make_async_copy(ones_v, cnt_s.at[dst_v.at[0]], csem).wait()
      return carry

    lax.fori_loop(0, NCH // 2, c_drain, 0)

    plsc.subcore_barrier()
    pltpu.sync_copy(acc_s.at[pl.ds(base, RPT)], sums_h.at[cid, pl.ds(base, RPT)])
    pltpu.sync_copy(cnt_s.at[pl.ds(base, RPT)], cnts_h.at[cid, pl.ds(base, RPT)])

  return k(table2, src2, dst3)


def _tc_mean_linear(sums, cnts, W, b, lrelu, split_out):
  def body(s_ref, c_ref, w_ref, b_ref, o_ref):
    s = jnp.concatenate([s_ref[0], s_ref[1]], axis=1)
    c = c_ref[0, :, 0:1] + c_ref[1, :, 0:1]
    agg = s / jnp.maximum(c, 1.0)
    h = jnp.dot(agg, w_ref[...], preferred_element_type=jnp.float32)
    h = h + b_ref[...] * (c > 0.0).astype(jnp.float32)
    if lrelu:
      h = jnp.where(h >= 0.0, h, 0.01 * h)
    if split_out:
      o_ref[0] = h[:, :DH]
      o_ref[1] = h[:, DH:]
    else:
      o_ref[...] = h

  if split_out:
    out_spec = pl.BlockSpec((NC, BR, DH), lambda i: (0, i, 0))
    out_shape = jax.ShapeDtypeStruct((NC, N, DH), jnp.float32)
  else:
    out_spec = pl.BlockSpec((BR, D), lambda i: (i, 0))
    out_shape = jax.ShapeDtypeStruct((N, D), jnp.float32)

  return pl.pallas_call(
      body,
      grid=(N // BR,),
      in_specs=[
          pl.BlockSpec((NC, BR, DH), lambda i: (0, i, 0)),
          pl.BlockSpec((NC, BR, CW), lambda i: (0, i, 0)),
          pl.BlockSpec((D, D), lambda i: (0, 0)),
          pl.BlockSpec((1, D), lambda i: (0, 0)),
      ],
      out_specs=out_spec,
      out_shape=out_shape,
  )(sums, cnts, W, b.reshape(1, D))


def _tc_mean_linear_head(sums, cnts, W1, b1, Wp, bp):
  def body(s_ref, c_ref, w1_ref, b1_ref, wp_ref, bp_ref, o_ref, h_ref):
    s = jnp.concatenate([s_ref[0], s_ref[1]], axis=1)
    c = c_ref[0, :, 0:1] + c_ref[1, :, 0:1]
    agg = s / jnp.maximum(c, 1.0)
    h = jnp.dot(agg, w1_ref[...], preferred_element_type=jnp.float32)
    h = h + b1_ref[...] * (c > 0.0).astype(jnp.float32)
    h_ref[...] = h
    o_ref[...] = (
        jnp.dot(h, wp_ref[...], preferred_element_type=jnp.float32)
        + bp_ref[...]
    )

  return pl.pallas_call(
      body,
      grid=(N // BR,),
      in_specs=[
          pl.BlockSpec((NC, BR, DH), lambda i: (0, i, 0)),
          pl.BlockSpec((NC, BR, CW), lambda i: (0, i, 0)),
          pl.BlockSpec((D, D), lambda i: (0, 0)),
          pl.BlockSpec((1, D), lambda i: (0, 0)),
          pl.BlockSpec((D, DO), lambda i: (0, 0)),
          pl.BlockSpec((1, DO), lambda i: (0, 0)),
      ],
      out_specs=[
          pl.BlockSpec((BR, DO), lambda i: (i, 0)),
          pl.BlockSpec((BR, D), lambda i: (i, 0)),
      ],
      out_shape=[
          jax.ShapeDtypeStruct((N, DO), jnp.float32),
          jax.ShapeDtypeStruct((N, D), jnp.float32),
      ],
  )(sums, cnts, W1, b1.reshape(1, D), Wp, bp.reshape(1, DO))


def kernel(feat_user, feat_item, edge_ui, edge_iu,
           W_ui_0, b_ui_0, W_iu_0, b_iu_0,
           W_ui_1, b_ui_1, W_iu_1, b_iu_1, W_p, b_p):
  edge_ui = edge_ui.astype(jnp.int32)
  edge_iu = edge_iu.astype(jnp.int32)
  src_ui = edge_ui[0].reshape(NS, EPT)
  dst_ui = edge_ui[1].reshape(NS, NCH, CH)
  src_iu = edge_iu[0].reshape(NS, EPT)
  dst_iu = edge_iu[1].reshape(NS, NCH, CH)

  fu2 = jnp.stack([feat_user[:, :DH], feat_user[:, DH:]])

  sums1, cnts1 = _sc_segment_sum(fu2, src_ui, dst_ui)
  h_i2 = _tc_mean_linear(sums1, cnts1, W_ui_0, b_ui_0, lrelu=True,
                         split_out=True)

  sums2, cnts2 = _sc_segment_sum(h_i2, src_iu, dst_iu)
  out, h_u2 = _tc_mean_linear_head(sums2, cnts2, W_iu_1, b_iu_1, W_p, b_p)
  return (out, h_u2)


# speedup vs baseline: 1.2327x; 1.2327x over previous
import functools

import jax
import jax.numpy as jnp
from jax import lax
from jax.experimental import pallas as pl
from jax.experimental.pallas import tpu as pltpu
from jax.experimental.pallas import tpu_sc as plsc

N = 10000
NPAD = 10240
D = 128
DH = D // 2
DO = 64
E = 320000
NC, NS = 2, 16
EPT = E // NS
CH = 80
NCH = EPT // CH
RPT = NPAD // NS
CW = 16
BR = 1000
NBUF = 5
LEAD = 4
DRAIN = NBUF - LEAD


def _fill2d(ref, rows, cols, value):
  vecs = cols // 16

  def body(t, carry):
    r = t // vecs
    c = (t % vecs) * 16
    ref[r, pl.ds(c, 16)] = jnp.full((16,), value, jnp.float32)
    return carry

  lax.fori_loop(0, rows * vecs, body, 0)


def _sc_segment_sum(table2, src2, dst3):
  mesh = plsc.VectorSubcoreMesh(core_axis_name="c", subcore_axis_name="s")

  @functools.partial(
      pl.kernel,
      out_type=[
          jax.ShapeDtypeStruct((NC, NPAD, DH), jnp.float32),
          jax.ShapeDtypeStruct((NC, NPAD, CW), jnp.float32),
      ],
      mesh=mesh,
      compiler_params=pltpu.CompilerParams(use_tc_tiling_on_sc=False),
      scratch_types=[
          pltpu.VMEM((EPT,), jnp.int32),
          pltpu.VMEM((NCH, CH), jnp.int32),
          pltpu.VMEM((NBUF, CH, DH), jnp.float32),
          pltpu.VMEM((CH, CW), jnp.float32),
          pltpu.VMEM((CH, CW), jnp.float32),
          pltpu.VMEM_SHARED((NPAD, DH), jnp.float32),
          pltpu.VMEM_SHARED((NPAD, CW), jnp.float32),
          pltpu.SemaphoreType.DMA((NBUF,)),
          pltpu.SemaphoreType.DMA((NBUF,)),
          pltpu.SemaphoreType.DMA,
      ],
  )
  def k(table_h, src_h, dst_h,
        sums_h, cnts_h,
        src_v, dst_v, rows_v, ones_v, zcnt_v, acc_s, cnt_s,
        gsem, ssem, csem):
    cid = lax.axis_index("c")
    sid = lax.axis_index("s")

    pltpu.sync_copy(src_h.at[sid], src_v)
    pltpu.sync_copy(dst_h.at[sid], dst_v)
    _fill2d(rows_v.at[0], CH, DH, 0.0)
    _fill2d(ones_v, CH, CW, 1.0)
    _fill2d(zcnt_v, CH, CW, 0.0)
    base = pl.multiple_of(sid * RPT, 8)
    for kk in range(RPT // CH):
      off = pl.multiple_of(base + kk * CH, 8)
      pltpu.sync_copy(rows_v.at[0], acc_s.at[pl.ds(off, CH)])
      pltpu.sync_copy(zcnt_v, cnt_s.at[pl.ds(off, CH)])

    my_table = table_h.at[cid]

    def g_issue(j, b):
      off = pl.multiple_of(j * CH, CH)
      pltpu.async_copy(my_table.at[src_v.at[pl.ds(off, CH)]], rows_v.at[b],
                       gsem.at[b])

    def g_wait(j, b):
      off = pl.multiple_of(j * CH, CH)
      pltpu.make_async_copy(my_table.at[src_v.at[pl.ds(off, CH)]],
                            rows_v.at[b], gsem.at[b]).wait()

    def s_issue(j, b):
      pltpu.async_copy(rows_v.at[b], acc_s.at[dst_v.at[j]], ssem.at[b],
                       add=True)

    def s_wait(j, b):
      pltpu.make_async_copy(rows_v.at[b], acc_s.at[dst_v.at[j]],
                            ssem.at[b]).wait()

    def c_issue(j):
      @pl.when(jnp.logical_xor(cid == 1, j < NCH // 2))
      def _():
        pltpu.async_copy(ones_v, cnt_s.at[dst_v.at[j]], csem, add=True)

    def step(j, b):
      g_wait(j, b)
      s_issue(j, b)
      c_issue(j)

    for b in range(LEAD):
      g_issue(b, b)
    plsc.subcore_barrier()

    for b in range(NBUF):
      j = b
      step(j, b)
      if j >= DRAIN:
        s_wait(j - DRAIN, (j + LEAD) % NBUF)
      g_issue(j + LEAD, (j + LEAD) % NBUF)

    def outer(go, carry):
      j0 = go * NBUF
      for b in range(NBUF):
        j = j0 + b
        step(j, b)
        s_wait(j - DRAIN, (b + LEAD) % NBUF)
        g_issue(j + LEAD, (b + LEAD) % NBUF)
      return carry

    lax.fori_loop(1, NCH // NBUF - 1, outer, 0)

    for b in range(NBUF):
      j = NCH - NBUF + b
      step(j, b)
      if j + LEAD < NCH:
        s_wait(j - DRAIN, (b + LEAD) % NBUF)
        g_issue(j + LEAD, (b + LEAD) % NBUF)
    for b in range(NBUF):
      j = NCH - NBUF + b
      s_wait(j, b)

    def c_drain(i, carry):
      pltpu.make_async_copy(ones_v, cnt_s.at[dst_v.at[0]], csem).wait()
      return carry

    lax.fori_loop(0, NCH // 2, c_drain, 0)

    plsc.subcore_barrier()
    pltpu.sync_copy(acc_s.at[pl.ds(base, RPT)], sums_h.at[cid, pl.ds(base, RPT)])
    pltpu.sync_copy(cnt_s.at[pl.ds(base, RPT)], cnts_h.at[cid, pl.ds(base, RPT)])

  return k(table2, src2, dst3)


def _tc_mean_linear(sums, cnts, W, b, lrelu, split_out):
  def body(s_ref, c_ref, w_ref, b_ref, o_ref):
    s = jnp.concatenate([s_ref[0], s_ref[1]], axis=1)
    c = c_ref[0, :, 0:1] + c_ref[1, :, 0:1]
    agg = s / jnp.maximum(c, 1.0)
    h = jnp.dot(agg, w_ref[...], preferred_element_type=jnp.float32)
    h = h + b_ref[...] * (c > 0.0).astype(jnp.float32)
    if lrelu:
      h = jnp.where(h >= 0.0, h, 0.01 * h)
    if split_out:
      o_ref[0] = h[:, :DH]
      o_ref[1] = h[:, DH:]
    else:
      o_ref[...] = h

  if split_out:
    out_spec = pl.BlockSpec((NC, BR, DH), lambda i: (0, i, 0))
    out_shape = jax.ShapeDtypeStruct((NC, N, DH), jnp.float32)
  else:
    out_spec = pl.BlockSpec((BR, D), lambda i: (i, 0))
    out_shape = jax.ShapeDtypeStruct((N, D), jnp.float32)

  return pl.pallas_call(
      body,
      grid=(N // BR,),
      in_specs=[
          pl.BlockSpec((NC, BR, DH), lambda i: (0, i, 0)),
          pl.BlockSpec((NC, BR, CW), lambda i: (0, i, 0)),
          pl.BlockSpec((D, D), lambda i: (0, 0)),
          pl.BlockSpec((1, D), lambda i: (0, 0)),
      ],
      out_specs=out_spec,
      out_shape=out_shape,
  )(sums, cnts, W, b.reshape(1, D))


def _tc_mean_linear_head(sums, cnts, W1, b1, Wp, bp):
  def body(s_ref, c_ref, w1_ref, b1_ref, wp_ref, bp_ref, o_ref, h_ref):
    s = jnp.concatenate([s_ref[0], s_ref[1]], axis=1)
    c = c_ref[0, :, 0:1] + c_ref[1, :, 0:1]
    agg = s / jnp.maximum(c, 1.0)
    h = jnp.dot(agg, w1_ref[...], preferred_element_type=jnp.float32)
    h = h + b1_ref[...] * (c > 0.0).astype(jnp.float32)
    h_ref[...] = h
    o_ref[...] = (
        jnp.dot(h, wp_ref[...], preferred_element_type=jnp.float32)
        + bp_ref[...]
    )

  return pl.pallas_call(
      body,
      grid=(N // BR,),
      in_specs=[
          pl.BlockSpec((NC, BR, DH), lambda i: (0, i, 0)),
          pl.BlockSpec((NC, BR, CW), lambda i: (0, i, 0)),
          pl.BlockSpec((D, D), lambda i: (0, 0)),
          pl.BlockSpec((1, D), lambda i: (0, 0)),
          pl.BlockSpec((D, DO), lambda i: (0, 0)),
          pl.BlockSpec((1, DO), lambda i: (0, 0)),
      ],
      out_specs=[
          pl.BlockSpec((BR, DO), lambda i: (i, 0)),
          pl.BlockSpec((BR, D), lambda i: (i, 0)),
      ],
      out_shape=[
          jax.ShapeDtypeStruct((N, DO), jnp.float32),
          jax.ShapeDtypeStruct((N, D), jnp.float32),
      ],
  )(sums, cnts, W1, b1.reshape(1, D), Wp, bp.reshape(1, DO))


def kernel(feat_user, feat_item, edge_ui, edge_iu,
           W_ui_0, b_ui_0, W_iu_0, b_iu_0,
           W_ui_1, b_ui_1, W_iu_1, b_iu_1, W_p, b_p):
  edge_ui = edge_ui.astype(jnp.int32)
  edge_iu = edge_iu.astype(jnp.int32)
  src_ui = edge_ui[0].reshape(NS, EPT)
  dst_ui = edge_ui[1].reshape(NS, NCH, CH)
  src_iu = edge_iu[0].reshape(NS, EPT)
  dst_iu = edge_iu[1].reshape(NS, NCH, CH)

  fu2 = jnp.stack([feat_user[:, :DH], feat_user[:, DH:]])

  sums1, cnts1 = _sc_segment_sum(fu2, src_ui, dst_ui)
  h_i2 = _tc_mean_linear(sums1, cnts1, W_ui_0, b_ui_0, lrelu=True,
                         split_out=True)

  sums2, cnts2 = _sc_segment_sum(h_i2, src_iu, dst_iu)
  out, h_u2 = _tc_mean_linear_head(sums2, cnts2, W_iu_1, b_iu_1, W_p, b_p)
  return (out, h_u2)


# speedup vs baseline: 1.2443x; 1.0095x over previous
import functools

import jax
import jax.numpy as jnp
from jax import lax
from jax.experimental import pallas as pl
from jax.experimental.pallas import tpu as pltpu
from jax.experimental.pallas import tpu_sc as plsc

N = 10000
NPAD = 10240
D = 128
DH = D // 2
DO = 64
E = 320000
NC, NS = 2, 16
EPT = E // NS
CH = 80
NCH = EPT // CH
RPT = NPAD // NS
CW = 16
BR = 2000
NBUF = 5
LEAD = 4
DRAIN = NBUF - LEAD


def _fill2d(ref, rows, cols, value):
  vecs = cols // 16

  def body(t, carry):
    r = t // vecs
    c = (t % vecs) * 16
    ref[r, pl.ds(c, 16)] = jnp.full((16,), value, jnp.float32)
    return carry

  lax.fori_loop(0, rows * vecs, body, 0)


def _sc_segment_sum(table2, src2, dst3):
  mesh = plsc.VectorSubcoreMesh(core_axis_name="c", subcore_axis_name="s")

  @functools.partial(
      pl.kernel,
      out_type=[
          jax.ShapeDtypeStruct((NC, NPAD, DH), jnp.float32),
          jax.ShapeDtypeStruct((NC, NPAD, CW), jnp.float32),
      ],
      mesh=mesh,
      compiler_params=pltpu.CompilerParams(use_tc_tiling_on_sc=False),
      scratch_types=[
          pltpu.VMEM((EPT,), jnp.int32),
          pltpu.VMEM((NCH, CH), jnp.int32),
          pltpu.VMEM((NBUF, CH, DH), jnp.float32),
          pltpu.VMEM((CH, CW), jnp.float32),
          pltpu.VMEM((CH, CW), jnp.float32),
          pltpu.VMEM_SHARED((NPAD, DH), jnp.float32),
          pltpu.VMEM_SHARED((NPAD, CW), jnp.float32),
          pltpu.SemaphoreType.DMA((NBUF,)),
          pltpu.SemaphoreType.DMA((NBUF,)),
          pltpu.SemaphoreType.DMA,
      ],
  )
  def k(table_h, src_h, dst_h,
        sums_h, cnts_h,
        src_v, dst_v, rows_v, ones_v, zcnt_v, acc_s, cnt_s,
        gsem, ssem, csem):
    cid = lax.axis_index("c")
    sid = lax.axis_index("s")

    pltpu.sync_copy(src_h.at[sid], src_v)
    pltpu.sync_copy(dst_h.at[sid], dst_v)
    _fill2d(rows_v.at[0], CH, DH, 0.0)
    _fill2d(ones_v, CH, CW, 1.0)
    _fill2d(zcnt_v, CH, CW, 0.0)
    base = pl.multiple_of(sid * RPT, 8)
    for kk in range(RPT // CH):
      off = pl.multiple_of(base + kk * CH, 8)
      pltpu.sync_copy(rows_v.at[0], acc_s.at[pl.ds(off, CH)])
      pltpu.sync_copy(zcnt_v, cnt_s.at[pl.ds(off, CH)])

    my_table = table_h.at[cid]

    def g_issue(j, b):
      off = pl.multiple_of(j * CH, CH)
      pltpu.async_copy(my_table.at[src_v.at[pl.ds(off, CH)]], rows_v.at[b],
                       gsem.at[b])

    def g_wait(j, b):
      off = pl.multiple_of(j * CH, CH)
      pltpu.make_async_copy(my_table.at[src_v.at[pl.ds(off, CH)]],
                            rows_v.at[b], gsem.at[b]).wait()

    def s_issue(j, b):
      pltpu.async_copy(rows_v.at[b], acc_s.at[dst_v.at[j]], ssem.at[b],
                       add=True)

    def s_wait(j, b):
      pltpu.make_async_copy(rows_v.at[b], acc_s.at[dst_v.at[j]],
                            ssem.at[b]).wait()

    def c_issue(j):
      @pl.when(jnp.logical_xor(cid == 1, j < NCH // 2))
      def _():
        pltpu.async_copy(ones_v, cnt_s.at[dst_v.at[j]], csem, add=True)

    def step(j, b):
      g_wait(j, b)
      s_issue(j, b)
      c_issue(j)

    for b in range(LEAD):
      g_issue(b, b)
    plsc.subcore_barrier()

    for b in range(NBUF):
      j = b
      step(j, b)
      if j >= DRAIN:
        s_wait(j - DRAIN, (j + LEAD) % NBUF)
      g_issue(j + LEAD, (j + LEAD) % NBUF)

    def outer(go, carry):
      j0 = go * NBUF
      for b in range(NBUF):
        j = j0 + b
        step(j, b)
        s_wait(j - DRAIN, (b + LEAD) % NBUF)
        g_issue(j + LEAD, (b + LEAD) % NBUF)
      return carry

    lax.fori_loop(1, NCH // NBUF - 1, outer, 0)

    for b in range(NBUF):
      j = NCH - NBUF + b
      step(j, b)
      if j + LEAD < NCH:
        s_wait(j - DRAIN, (b + LEAD) % NBUF)
        g_issue(j + LEAD, (b + LEAD) % NBUF)
    for b in range(NBUF):
      j = NCH - NBUF + b
      s_wait(j, b)

    def c_drain(i, carry):
      pltpu.make_async_copy(ones_v, cnt_s.at[dst_v.at[0]], csem).wait()
      return carry

    lax.fori_loop(0, NCH // 2, c_drain, 0)

    plsc.subcore_barrier()
    pltpu.sync_copy(acc_s.at[pl.ds(base, RPT)], sums_h.at[cid, pl.ds(base, RPT)])
    pltpu.sync_copy(cnt_s.at[pl.ds(base, RPT)], cnts_h.at[cid, pl.ds(base, RPT)])

  return k(table2, src2, dst3)


def _tc_mean_linear(sums, cnts, W, b, lrelu, split_out):
  def body(s_ref, c_ref, w_ref, b_ref, o_ref):
    s = jnp.concatenate([s_ref[0], s_ref[1]], axis=1)
    c = c_ref[0, :, 0:1] + c_ref[1, :, 0:1]
    agg = s / jnp.maximum(c, 1.0)
    h = jnp.dot(agg, w_ref[...], preferred_element_type=jnp.float32)
    h = h + b_ref[...] * (c > 0.0).astype(jnp.float32)
    if lrelu:
      h = jnp.where(h >= 0.0, h, 0.01 * h)
    if split_out:
      o_ref[0] = h[:, :DH]
      o_ref[1] = h[:, DH:]
    else:
      o_ref[...] = h

  if split_out:
    out_spec = pl.BlockSpec((NC, BR, DH), lambda i: (0, i, 0))
    out_shape = jax.ShapeDtypeStruct((NC, N, DH), jnp.float32)
  else:
    out_spec = pl.BlockSpec((BR, D), lambda i: (i, 0))
    out_shape = jax.ShapeDtypeStruct((N, D), jnp.float32)

  return pl.pallas_call(
      body,
      grid=(N // BR,),
      in_specs=[
          pl.BlockSpec((NC, BR, DH), lambda i: (0, i, 0)),
          pl.BlockSpec((NC, BR, CW), lambda i: (0, i, 0)),
          pl.BlockSpec((D, D), lambda i: (0, 0)),
          pl.BlockSpec((1, D), lambda i: (0, 0)),
      ],
      out_specs=out_spec,
      out_shape=out_shape,
  )(sums, cnts, W, b.reshape(1, D))


def _tc_mean_linear_head(sums, cnts, W1, b1, Wp, bp):
  def body(s_ref, c_ref, w1_ref, b1_ref, wp_ref, bp_ref, o_ref, h_ref):
    s = jnp.concatenate([s_ref[0], s_ref[1]], axis=1)
    c = c_ref[0, :, 0:1] + c_ref[1, :, 0:1]
    agg = s / jnp.maximum(c, 1.0)
    h = jnp.dot(agg, w1_ref[...], preferred_element_type=jnp.float32)
    h = h + b1_ref[...] * (c > 0.0).astype(jnp.float32)
    h_ref[...] = h
    o_ref[...] = (
        jnp.dot(h, wp_ref[...], preferred_element_type=jnp.float32)
        + bp_ref[...]
    )

  return pl.pallas_call(
      body,
      grid=(N // BR,),
      in_specs=[
          pl.BlockSpec((NC, BR, DH), lambda i: (0, i, 0)),
          pl.BlockSpec((NC, BR, CW), lambda i: (0, i, 0)),
          pl.BlockSpec((D, D), lambda i: (0, 0)),
          pl.BlockSpec((1, D), lambda i: (0, 0)),
          pl.BlockSpec((D, DO), lambda i: (0, 0)),
          pl.BlockSpec((1, DO), lambda i: (0, 0)),
      ],
      out_specs=[
          pl.BlockSpec((BR, DO), lambda i: (i, 0)),
          pl.BlockSpec((BR, D), lambda i: (i, 0)),
      ],
      out_shape=[
          jax.ShapeDtypeStruct((N, DO), jnp.float32),
          jax.ShapeDtypeStruct((N, D), jnp.float32),
      ],
  )(sums, cnts, W1, b1.reshape(1, D), Wp, bp.reshape(1, DO))


def kernel(feat_user, feat_item, edge_ui, edge_iu,
           W_ui_0, b_ui_0, W_iu_0, b_iu_0,
           W_ui_1, b_ui_1, W_iu_1, b_iu_1, W_p, b_p):
  edge_ui = edge_ui.astype(jnp.int32)
  edge_iu = edge_iu.astype(jnp.int32)
  src_ui = edge_ui[0].reshape(NS, EPT)
  dst_ui = edge_ui[1].reshape(NS, NCH, CH)
  src_iu = edge_iu[0].reshape(NS, EPT)
  dst_iu = edge_iu[1].reshape(NS, NCH, CH)

  fu2 = jnp.stack([feat_user[:, :DH], feat_user[:, DH:]])

  sums1, cnts1 = _sc_segment_sum(fu2, src_ui, dst_ui)
  h_i2 = _tc_mean_linear(sums1, cnts1, W_ui_0, b_ui_0, lrelu=True,
                         split_out=True)

  sums2, cnts2 = _sc_segment_sum(h_i2, src_iu, dst_iu)
  out, h_u2 = _tc_mean_linear_head(sums2, cnts2, W_iu_1, b_iu_1, W_p, b_p)
  return (out, h_u2)


# speedup vs baseline: 1.2777x; 1.0268x over previous
import functools

import jax
import jax.numpy as jnp
from jax import lax
from jax.experimental import pallas as pl
from jax.experimental.pallas import tpu as pltpu
from jax.experimental.pallas import tpu_sc as plsc

N = 10000
NPAD = 10240
D = 128
DH = D // 2
DO = 64
E = 320000
NC, NS = 2, 16
EPT = E // NS
CH = 80
NCH = EPT // CH
RPT = NPAD // NS
CW = 16
BR = 2000
NBUF = 5
LEAD = 4
DRAIN = NBUF - LEAD


def _fill2d(ref, rows, cols, value):
  vecs = cols // 16

  def body(t, carry):
    r = t // vecs
    c = (t % vecs) * 16
    ref[r, pl.ds(c, 16)] = jnp.full((16,), value, jnp.float32)
    return carry

  lax.fori_loop(0, rows * vecs, body, 0)


def _sc_segment_sum(table2, src2, dst3):
  mesh = plsc.VectorSubcoreMesh(core_axis_name="c", subcore_axis_name="s")

  @functools.partial(
      pl.kernel,
      out_type=[
          jax.ShapeDtypeStruct((NC, NPAD, DH), jnp.float32),
          jax.ShapeDtypeStruct((NC, NPAD, CW), jnp.float32),
      ],
      mesh=mesh,
      compiler_params=pltpu.CompilerParams(use_tc_tiling_on_sc=False),
      scratch_types=[
          pltpu.VMEM((EPT,), jnp.int32),
          pltpu.VMEM((NCH, CH), jnp.int32),
          pltpu.VMEM((NBUF, CH, DH), jnp.float32),
          pltpu.VMEM((CH, CW), jnp.float32),
          pltpu.VMEM((CH, CW), jnp.float32),
          pltpu.VMEM_SHARED((NPAD, DH), jnp.float32),
          pltpu.VMEM_SHARED((NPAD, CW), jnp.float32),
          pltpu.SemaphoreType.DMA((NBUF,)),
          pltpu.SemaphoreType.DMA((NBUF,)),
          pltpu.SemaphoreType.DMA,
      ],
  )
  def k(table_h, src_h, dst_h,
        sums_h, cnts_h,
        src_v, dst_v, rows_v, ones_v, zcnt_v, acc_s, cnt_s,
        gsem, ssem, csem):
    cid = lax.axis_index("c")
    sid = lax.axis_index("s")

    pltpu.async_copy(src_h.at[sid], src_v, csem)
    pltpu.async_copy(dst_h.at[sid], dst_v, csem)
    _fill2d(rows_v.at[0], CH, DH, 0.0)
    _fill2d(ones_v, CH, CW, 1.0)
    _fill2d(zcnt_v, CH, CW, 0.0)
    base = pl.multiple_of(sid * RPT, 8)
    for kk in range(RPT // CH):
      off = pl.multiple_of(base + kk * CH, 8)
      pltpu.async_copy(rows_v.at[0], acc_s.at[pl.ds(off, CH)], csem)
      pltpu.async_copy(zcnt_v, cnt_s.at[pl.ds(off, CH)], csem)
    pltpu.make_async_copy(src_h.at[sid], src_v, csem).wait()
    pltpu.make_async_copy(dst_h.at[sid], dst_v, csem).wait()
    for kk in range(RPT // CH):
      off = pl.multiple_of(base + kk * CH, 8)
      pltpu.make_async_copy(rows_v.at[0], acc_s.at[pl.ds(off, CH)], csem).wait()
      pltpu.make_async_copy(zcnt_v, cnt_s.at[pl.ds(off, CH)], csem).wait()

    my_table = table_h.at[cid]

    def g_issue(j, b):
      off = pl.multiple_of(j * CH, CH)
      pltpu.async_copy(my_table.at[src_v.at[pl.ds(off, CH)]], rows_v.at[b],
                       gsem.at[b])

    def g_wait(j, b):
      off = pl.multiple_of(j * CH, CH)
      pltpu.make_async_copy(my_table.at[src_v.at[pl.ds(off, CH)]],
                            rows_v.at[b], gsem.at[b]).wait()

    def s_issue(j, b):
      pltpu.async_copy(rows_v.at[b], acc_s.at[dst_v.at[j]], ssem.at[b],
                       add=True)

    def s_wait(j, b):
      pltpu.make_async_copy(rows_v.at[b], acc_s.at[dst_v.at[j]],
                            ssem.at[b]).wait()

    def c_issue(j):
      @pl.when(jnp.logical_xor(cid == 1, j < NCH // 2))
      def _():
        pltpu.async_copy(ones_v, cnt_s.at[dst_v.at[j]], csem, add=True)

    def step(j, b):
      g_wait(j, b)
      s_issue(j, b)
      c_issue(j)

    for b in range(LEAD):
      g_issue(b, b)
    plsc.subcore_barrier()

    for b in range(NBUF):
      j = b
      step(j, b)
      if j >= DRAIN:
        s_wait(j - DRAIN, (j + LEAD) % NBUF)
      g_issue(j + LEAD, (j + LEAD) % NBUF)

    def outer(go, carry):
      j0 = go * NBUF
      for b in range(NBUF):
        j = j0 + b
        step(j, b)
        s_wait(j - DRAIN, (b + LEAD) % NBUF)
        g_issue(j + LEAD, (b + LEAD) % NBUF)
      return carry

    lax.fori_loop(1, NCH // NBUF - 1, outer, 0)

    for b in range(NBUF):
      j = NCH - NBUF + b
      step(j, b)
      if j + LEAD < NCH:
        s_wait(j - DRAIN, (b + LEAD) % NBUF)
        g_issue(j + LEAD, (b + LEAD) % NBUF)
    for b in range(NBUF):
      j = NCH - NBUF + b
      s_wait(j, b)

    def c_drain(i, carry):
      pltpu.make_async_copy(ones_v, cnt_s.at[dst_v.at[0]], csem).wait()
      return carry

    lax.fori_loop(0, NCH // 2, c_drain, 0)

    plsc.subcore_barrier()
    pltpu.sync_copy(acc_s.at[pl.ds(base, RPT)], sums_h.at[cid, pl.ds(base, RPT)])
    pltpu.sync_copy(cnt_s.at[pl.ds(base, RPT)], cnts_h.at[cid, pl.ds(base, RPT)])

  return k(table2, src2, dst3)


def _tc_mean_linear(sums, cnts, W, b, lrelu, split_out):
  def body(s_ref, c_ref, w_ref, b_ref, o_ref):
    s = jnp.concatenate([s_ref[0], s_ref[1]], axis=1)
    c = c_ref[0, :, 0:1] + c_ref[1, :, 0:1]
    agg = s / jnp.maximum(c, 1.0)
    h = jnp.dot(agg, w_ref[...], preferred_element_type=jnp.float32)
    h = h + b_ref[...] * (c > 0.0).astype(jnp.float32)
    if lrelu:
      h = jnp.where(h >= 0.0, h, 0.01 * h)
    if split_out:
      o_ref[0] = h[:, :DH]
      o_ref[1] = h[:, DH:]
    else:
      o_ref[...] = h

  if split_out:
    out_spec = pl.BlockSpec((NC, BR, DH), lambda i: (0, i, 0))
    out_shape = jax.ShapeDtypeStruct((NC, N, DH), jnp.float32)
  else:
    out_spec = pl.BlockSpec((BR, D), lambda i: (i, 0))
    out_shape = jax.ShapeDtypeStruct((N, D), jnp.float32)

  return pl.pallas_call(
      body,
      grid=(N // BR,),
      in_specs=[
          pl.BlockSpec((NC, BR, DH), lambda i: (0, i, 0)),
          pl.BlockSpec((NC, BR, CW), lambda i: (0, i, 0)),
          pl.BlockSpec((D, D), lambda i: (0, 0)),
          pl.BlockSpec((1, D), lambda i: (0, 0)),
      ],
      out_specs=out_spec,
      out_shape=out_shape,
  )(sums, cnts, W, b.reshape(1, D))


def _tc_mean_linear_head(sums, cnts, W1, b1, Wp, bp):
  def body(s_ref, c_ref, w1_ref, b1_ref, wp_ref, bp_ref, o_ref, h_ref):
    s = jnp.concatenate([s_ref[0], s_ref[1]], axis=1)
    c = c_ref[0, :, 0:1] + c_ref[1, :, 0:1]
    agg = s / jnp.maximum(c, 1.0)
    h = jnp.dot(agg, w1_ref[...], preferred_element_type=jnp.float32)
    h = h + b1_ref[...] * (c > 0.0).astype(jnp.float32)
    h_ref[...] = h
    o_ref[...] = (
        jnp.dot(h, wp_ref[...], preferred_element_type=jnp.float32)
        + bp_ref[...]
    )

  return pl.pallas_call(
      body,
      grid=(N // BR,),
      in_specs=[
          pl.BlockSpec((NC, BR, DH), lambda i: (0, i, 0)),
          pl.BlockSpec((NC, BR, CW), lambda i: (0, i, 0)),
          pl.BlockSpec((D, D), lambda i: (0, 0)),
          pl.BlockSpec((1, D), lambda i: (0, 0)),
          pl.BlockSpec((D, DO), lambda i: (0, 0)),
          pl.BlockSpec((1, DO), lambda i: (0, 0)),
      ],
      out_specs=[
          pl.BlockSpec((BR, DO), lambda i: (i, 0)),
          pl.BlockSpec((BR, D), lambda i: (i, 0)),
      ],
      out_shape=[
          jax.ShapeDtypeStruct((N, DO), jnp.float32),
          jax.ShapeDtypeStruct((N, D), jnp.float32),
      ],
  )(sums, cnts, W1, b1.reshape(1, D), Wp, bp.reshape(1, DO))


def kernel(feat_user, feat_item, edge_ui, edge_iu,
           W_ui_0, b_ui_0, W_iu_0, b_iu_0,
           W_ui_1, b_ui_1, W_iu_1, b_iu_1, W_p, b_p):
  edge_ui = edge_ui.astype(jnp.int32)
  edge_iu = edge_iu.astype(jnp.int32)
  src_ui = edge_ui[0].reshape(NS, EPT)
  dst_ui = edge_ui[1].reshape(NS, NCH, CH)
  src_iu = edge_iu[0].reshape(NS, EPT)
  dst_iu = edge_iu[1].reshape(NS, NCH, CH)

  fu2 = jnp.stack([feat_user[:, :DH], feat_user[:, DH:]])

  sums1, cnts1 = _sc_segment_sum(fu2, src_ui, dst_ui)
  h_i2 = _tc_mean_linear(sums1, cnts1, W_ui_0, b_ui_0, lrelu=True,
                         split_out=True)

  sums2, cnts2 = _sc_segment_sum(h_i2, src_iu, dst_iu)
  out, h_u2 = _tc_mean_linear_head(sums2, cnts2, W_iu_1, b_iu_1, W_p, b_p)
  return (out, h_u2)
